# baseline (speedup 1.0000x reference)
import jax
import jax.numpy as jnp
from jax import lax
from jax.experimental import pallas as pl
from jax.experimental.pallas import tpu as pltpu
from jax.experimental.pallas import tpu_sc as plsc

Z_N = 16
BATCH = 16384
HIST = 200
B_TOTAL = BATCH * HIST

NUM_CORES = 2
NUM_SUBCORES = 16
NUM_WORKERS = NUM_CORES * NUM_SUBCORES

IDX_W = 128
CHUNK = 1280
SUB = CHUNK // IDX_W
B_PER_W = B_TOTAL // NUM_WORKERS
N_CHUNKS = B_PER_W // CHUNK
PAIRS = N_CHUNKS // 2


def _gather_kernel(x_hbm, tbl_hbm, out_mu, out_lv,
                   idx0, idx1, rows0, rows1,
                   sg0, sg1, so0, so1):
    wid = lax.axis_index("s") * NUM_CORES + lax.axis_index("c")
    base = wid * B_PER_W
    row_base = wid * (B_PER_W // IDX_W)
    last = N_CHUNKS - 1

    slots = ((idx0, rows0, sg0, so0),
             (idx1, rows1, sg1, so1))

    def fire_gathers(c, slot):
        idx, rows, sg, _ = slots[slot]
        pltpu.sync_copy(x_hbm.at[pl.ds(row_base + c * SUB, SUB)], idx)
        for j in range(SUB):
            pltpu.async_copy(tbl_hbm.at[idx.at[j]],
                             rows.at[pl.ds(j * IDX_W, IDX_W)], sg)

    def wait_gathers(slot):
        idx, rows, sg, _ = slots[slot]
        pltpu.make_async_copy(tbl_hbm.at[pl.ds(0, CHUNK)], rows, sg).wait()

    def fire_writes(c, slot):
        _, rows, _, so = slots[slot]
        start = base + c * CHUNK
        pltpu.async_copy(rows.at[:, pl.ds(0, Z_N)],
                         out_mu.at[pl.ds(start, CHUNK)], so)
        pltpu.async_copy(rows.at[:, pl.ds(Z_N, Z_N)],
                         out_lv.at[pl.ds(start, CHUNK)], so)

    def wait_writes(slot):
        _, rows, _, so = slots[slot]
        pltpu.make_async_copy(rows.at[:, pl.ds(0, Z_N)],
                              out_mu.at[pl.ds(0, CHUNK)], so).wait()
        pltpu.make_async_copy(rows.at[:, pl.ds(Z_N, Z_N)],
                              out_lv.at[pl.ds(0, CHUNK)], so).wait()

    fire_gathers(0, 0)
    fire_gathers(1, 1)

    def body(g, carry):
        c0 = 2 * g
        c1 = c0 + 1
        wait_gathers(0)
        fire_writes(c0, 0)
        wait_gathers(1)
        fire_writes(c1, 1)
        n0 = jnp.minimum(c0 + 2, last)
        n1 = jnp.minimum(c1 + 2, last)
        wait_writes(0)
        fire_gathers(n0, 0)
        wait_writes(1)
        fire_gathers(n1, 1)
        return carry

    lax.fori_loop(0, PAIRS, body, 0)
    wait_gathers(0)
    wait_gathers(1)


@jax.jit
def kernel(x, embed_mu, embed_logvar):
    x_flat = x.reshape(B_TOTAL // IDX_W, IDX_W).astype(jnp.int32)
    tbl = jnp.concatenate([embed_mu, embed_logvar], axis=1)

    mesh = plsc.VectorSubcoreMesh(core_axis_name="c", subcore_axis_name="s")
    f = pl.kernel(
        _gather_kernel,
        out_type=(
            jax.ShapeDtypeStruct((B_TOTAL, Z_N), jnp.float32),
            jax.ShapeDtypeStruct((B_TOTAL, Z_N), jnp.float32),
        ),
        mesh=mesh,
        scratch_types=[
            pltpu.VMEM((SUB, IDX_W), jnp.int32),
            pltpu.VMEM((SUB, IDX_W), jnp.int32),
            pltpu.VMEM((CHUNK, 2 * Z_N), jnp.float32),
            pltpu.VMEM((CHUNK, 2 * Z_N), jnp.float32),
            pltpu.SemaphoreType.DMA,
            pltpu.SemaphoreType.DMA,
            pltpu.SemaphoreType.DMA,
            pltpu.SemaphoreType.DMA,
        ],
        compiler_params=pltpu.CompilerParams(use_tc_tiling_on_sc=False),
    )
    mu_flat, lv_flat = f(x_flat, tbl)
    mu = mu_flat.reshape(BATCH, HIST, Z_N)
    logvar = lv_flat.reshape(BATCH, HIST, Z_N)
    return (mu, mu, logvar)
